# baseline (speedup 1.0000x reference)
import functools

import jax
import jax.numpy as jnp
from jax import lax
from jax.experimental import pallas as pl
from jax.experimental.pallas import tpu as pltpu
from jax.experimental.pallas import tpu_sc as plsc

N_SCAN = 50000
M_TMPL = 8192
NPAD = 51200
NROWS = NPAD // 128
NT = 40
NSTEPS = NROWS // NT
BM = 8
PAD_VAL = 1e18
BIG_I = 2147483647
RM2 = 512

NUM_CORES = 2
NUM_SUBCORES = 16
NUM_TILES = NUM_CORES * NUM_SUBCORES
PER_TILE = M_TMPL // NUM_TILES
LANES = 16


def _ssq_body(scan_ref, out_ref):
    s0 = scan_ref[0]
    s1 = scan_ref[1]
    s2 = scan_ref[2]
    out_ref[...] = s0 * s0 + s1 * s1 + s2 * s2


def _ssq_call(scan3):
    return pl.pallas_call(
        _ssq_body,
        out_shape=jax.ShapeDtypeStruct((NROWS, 128), jnp.float32),
    )(scan3)


def _nn_body(tmpl_ref, scan_ref, ssq_ref, outd_ref, outi_ref):
    a0 = [tmpl_ref[m, 0] * -2.0 for m in range(BM)]
    a1 = [tmpl_ref[m, 1] * -2.0 for m in range(BM)]
    a2 = [tmpl_ref[m, 2] * -2.0 for m in range(BM)]

    def step(r, carry):
        bds, bis = carry
        base = pl.multiple_of(r * NT, NT)
        s0 = scan_ref[0, pl.ds(base, NT), :]
        s1 = scan_ref[1, pl.ds(base, NT), :]
        s2 = scan_ref[2, pl.ds(base, NT), :]
        sq = ssq_ref[pl.ds(base, NT), :]
        g0 = ((NT // 8) * r).astype(jnp.int32)
        bds_n, bis_n = [], []
        for m in range(BM):
            d = sq + a0[m] * s0
            d = d + a1[m] * s1
            d = d + a2[m] * s2
            bd, bi = bds[m], bis[m]
            for j in range(NT // 8):
                dj = d[8 * j:8 * j + 8, :]
                upd = dj < bd
                bi = jnp.where(upd, g0 + j, bi)
                bd = jnp.minimum(dj, bd)
            bds_n.append(bd)
            bis_n.append(bi)
        return tuple(bds_n), tuple(bis_n)

    bd0 = tuple(jnp.full((8, 128), jnp.inf, jnp.float32) for _ in range(BM))
    bi0 = tuple(jnp.zeros((8, 128), jnp.int32) for _ in range(BM))
    bds, bis = lax.fori_loop(0, NSTEPS, step, (bd0, bi0))

    sub_l = lax.broadcasted_iota(jnp.int32, (8, 128), 0) * 128
    lane = lax.broadcasted_iota(jnp.int32, (8, 128), 1)
    rows_d, rows_i = [], []
    for m in range(BM):
        bd, bi = bds[m], bis[m]
        n_idx = bi * 1024 + sub_l + lane
        dmin = jnp.min(bd, axis=0, keepdims=True)
        cand = jnp.where(bd == dmin, n_idx, BIG_I)
        rows_d.append(dmin)
        rows_i.append(jnp.min(cand, axis=0, keepdims=True))
    outd_ref[...] = jnp.concatenate(rows_d, axis=0)
    outi_ref[...] = jnp.concatenate(rows_i, axis=0)


def _nn_call(tmpl, scan3, ssq):
    return pl.pallas_call(
        _nn_body,
        grid=(M_TMPL // BM,),
        in_specs=[
            pl.BlockSpec((BM, 3), lambda i: (i, 0),
                         memory_space=pltpu.SMEM),
            pl.BlockSpec((3, NROWS, 128), lambda i: (0, 0, 0)),
            pl.BlockSpec((NROWS, 128), lambda i: (0, 0)),
        ],
        out_specs=[
            pl.BlockSpec((BM, 128), lambda i: (i, 0)),
            pl.BlockSpec((BM, 128), lambda i: (i, 0)),
        ],
        out_shape=[
            jax.ShapeDtypeStruct((M_TMPL, 128), jnp.float32),
            jax.ShapeDtypeStruct((M_TMPL, 128), jnp.int32),
        ],
        compiler_params=pltpu.CompilerParams(
            dimension_semantics=("parallel",)),
    )(tmpl, scan3, ssq)


def _reduce_body(df_ref, if_ref, tmpl_ref, outd_ref, outi_ref):
    d = df_ref[...]
    i = if_ref[...]
    t = tmpl_ref[...]
    rmin = jnp.min(d, axis=1, keepdims=True)
    cand = jnp.where(d == rmin, i, BIG_I)
    imin = jnp.min(cand, axis=1, keepdims=True)
    tt = t[:, 0:1] * t[:, 0:1] + t[:, 1:2] * t[:, 1:2] + t[:, 2:3] * t[:, 2:3]
    outd_ref[...] = rmin + tt
    outi_ref[...] = imin


def _reduce_call(dfold, ifold, tmpl):
    return pl.pallas_call(
        _reduce_body,
        grid=(M_TMPL // RM2,),
        in_specs=[
            pl.BlockSpec((RM2, 128), lambda i: (i, 0)),
            pl.BlockSpec((RM2, 128), lambda i: (i, 0)),
            pl.BlockSpec((RM2, 3), lambda i: (i, 0)),
        ],
        out_specs=[
            pl.BlockSpec((RM2, 1), lambda i: (i, 0)),
            pl.BlockSpec((RM2, 1), lambda i: (i, 0)),
        ],
        out_shape=[
            jax.ShapeDtypeStruct((M_TMPL, 1), jnp.float32),
            jax.ShapeDtypeStruct((M_TMPL, 1), jnp.int32),
        ],
        compiler_params=pltpu.CompilerParams(
            dimension_semantics=("parallel",)),
    )(dfold, ifold, tmpl)


def _sc_gather_finish(dists, idx, snx, sny, snz, tnx, tny, tnz):
    mesh = plsc.VectorSubcoreMesh(
        core_axis_name="c", subcore_axis_name="s",
        num_cores=NUM_CORES, num_subcores=NUM_SUBCORES)

    @functools.partial(
        pl.kernel,
        out_type=jax.ShapeDtypeStruct((NUM_TILES, LANES), jnp.float32),
        mesh=mesh,
        scratch_types=[
            pltpu.VMEM((PER_TILE,), jnp.int32),
            pltpu.VMEM((PER_TILE,), jnp.float32),
            pltpu.VMEM((PER_TILE,), jnp.float32),
            pltpu.VMEM((PER_TILE,), jnp.float32),
            pltpu.VMEM((PER_TILE,), jnp.float32),
            pltpu.VMEM((PER_TILE,), jnp.float32),
            pltpu.VMEM((PER_TILE,), jnp.float32),
            pltpu.VMEM((PER_TILE,), jnp.float32),
            pltpu.VMEM((LANES,), jnp.float32),
            pltpu.SemaphoreType.DMA,
        ],
    )
    def sck(d_hbm, i_hbm, snx_hbm, sny_hbm, snz_hbm,
            tnx_hbm, tny_hbm, tnz_hbm, out_hbm,
            idx_v, d_v, gx_v, gy_v, gz_v, tx_v, ty_v, tz_v, acc_v, sem):
        wid = lax.axis_index("s") * NUM_CORES + lax.axis_index("c")
        base = wid * PER_TILE
        pltpu.sync_copy(i_hbm.at[pl.ds(base, PER_TILE)], idx_v)
        pltpu.sync_copy(d_hbm.at[pl.ds(base, PER_TILE)], d_v)
        pltpu.sync_copy(tnx_hbm.at[pl.ds(base, PER_TILE)], tx_v)
        pltpu.sync_copy(tny_hbm.at[pl.ds(base, PER_TILE)], ty_v)
        pltpu.sync_copy(tnz_hbm.at[pl.ds(base, PER_TILE)], tz_v)
        pltpu.async_copy(snx_hbm.at[idx_v], gx_v, sem).wait()
        pltpu.async_copy(sny_hbm.at[idx_v], gy_v, sem).wait()
        pltpu.async_copy(snz_hbm.at[idx_v], gz_v, sem).wait()
        acc = jnp.zeros((LANES,), jnp.float32)
        for j in range(PER_TILE // LANES):
            sl = pl.ds(j * LANES, LANES)
            dot = gx_v[sl] * tx_v[sl] + gy_v[sl] * ty_v[sl] + gz_v[sl] * tz_v[sl]
            acc = acc + jnp.where(dot > 0.5, d_v[sl], 0.0)
        acc_v[...] = acc
        pltpu.sync_copy(acc_v, out_hbm.at[wid])

    return sck(dists, idx, snx, sny, snz, tnx, tny, tnz)


def kernel(scan_vertices, template_vertices, scan_normals, template_normals):
    scan3 = jnp.pad(scan_vertices.T, ((0, 0), (0, NPAD - N_SCAN)),
                    constant_values=PAD_VAL).reshape(3, NROWS, 128)
    ssq = _ssq_call(scan3)
    dfold, ifold = _nn_call(template_vertices, scan3, ssq)
    d2, i2 = _reduce_call(dfold, ifold, template_vertices)
    dists = d2[:, 0]
    idx = i2[:, 0]
    snx, sny, snz = scan_normals[:, 0], scan_normals[:, 1], scan_normals[:, 2]
    tnx, tny, tnz = (template_normals[:, 0], template_normals[:, 1],
                     template_normals[:, 2])
    partials = _sc_gather_finish(dists, idx, snx, sny, snz, tnx, tny, tnz)
    return jnp.sum(partials)


# speedup vs baseline: 1.1032x; 1.1032x over previous
import functools

import jax
import jax.numpy as jnp
from jax import lax
from jax.experimental import pallas as pl
from jax.experimental.pallas import tpu as pltpu
from jax.experimental.pallas import tpu_sc as plsc

N_SCAN = 50000
M_TMPL = 8192
NPAD = 51200
NROWS = NPAD // 128
NT = 40
NSTEPS = NROWS // NT
BM = 8
PAD_VAL = 1e18
BIG_I = 2147483647
RM2 = 512

NUM_CORES = 2
NUM_SUBCORES = 16
NUM_TILES = NUM_CORES * NUM_SUBCORES
PER_TILE = M_TMPL // NUM_TILES
LANES = 16


def _ssq_body(scan_ref, out_ref):
    s0 = scan_ref[0]
    s1 = scan_ref[1]
    s2 = scan_ref[2]
    out_ref[...] = s0 * s0 + s1 * s1 + s2 * s2


def _ssq_call(scan3):
    return pl.pallas_call(
        _ssq_body,
        out_shape=jax.ShapeDtypeStruct((NROWS, 128), jnp.float32),
    )(scan3)


def _nn_body(tmpl_ref, scan_ref, ssq_ref, outd_ref, outi_ref):
    a0 = [tmpl_ref[m, 0] * -2.0 for m in range(BM)]
    a1 = [tmpl_ref[m, 1] * -2.0 for m in range(BM)]
    a2 = [tmpl_ref[m, 2] * -2.0 for m in range(BM)]

    bds = [jnp.full((8, 128), jnp.inf, jnp.float32) for _ in range(BM)]
    bis = [jnp.zeros((8, 128), jnp.int32) for _ in range(BM)]
    for r in range(NSTEPS):
        base = r * NT
        s0 = scan_ref[0, pl.ds(base, NT), :]
        s1 = scan_ref[1, pl.ds(base, NT), :]
        s2 = scan_ref[2, pl.ds(base, NT), :]
        sq = ssq_ref[pl.ds(base, NT), :]
        g0 = (NT // 8) * r
        for m in range(BM):
            d = sq + a0[m] * s0
            d = d + a1[m] * s1
            d = d + a2[m] * s2
            bd, bi = bds[m], bis[m]
            for j in range(NT // 8):
                dj = d[8 * j:8 * j + 8, :]
                upd = dj < bd
                bi = jnp.where(upd, g0 + j, bi)
                bd = jnp.minimum(dj, bd)
            bds[m] = bd
            bis[m] = bi

    sub_l = lax.broadcasted_iota(jnp.int32, (8, 128), 0) * 128
    lane = lax.broadcasted_iota(jnp.int32, (8, 128), 1)
    rows_d, rows_i = [], []
    for m in range(BM):
        bd, bi = bds[m], bis[m]
        n_idx = bi * 1024 + sub_l + lane
        dmin = jnp.min(bd, axis=0, keepdims=True)
        cand = jnp.where(bd == dmin, n_idx, BIG_I)
        rows_d.append(dmin)
        rows_i.append(jnp.min(cand, axis=0, keepdims=True))
    outd_ref[...] = jnp.concatenate(rows_d, axis=0)
    outi_ref[...] = jnp.concatenate(rows_i, axis=0)


def _nn_call(tmpl, scan3, ssq):
    return pl.pallas_call(
        _nn_body,
        grid=(M_TMPL // BM,),
        in_specs=[
            pl.BlockSpec((BM, 3), lambda i: (i, 0),
                         memory_space=pltpu.SMEM),
            pl.BlockSpec((3, NROWS, 128), lambda i: (0, 0, 0)),
            pl.BlockSpec((NROWS, 128), lambda i: (0, 0)),
        ],
        out_specs=[
            pl.BlockSpec((BM, 128), lambda i: (i, 0)),
            pl.BlockSpec((BM, 128), lambda i: (i, 0)),
        ],
        out_shape=[
            jax.ShapeDtypeStruct((M_TMPL, 128), jnp.float32),
            jax.ShapeDtypeStruct((M_TMPL, 128), jnp.int32),
        ],
        compiler_params=pltpu.CompilerParams(
            dimension_semantics=("parallel",)),
    )(tmpl, scan3, ssq)


def _reduce_body(df_ref, if_ref, tmpl_ref, outd_ref, outi_ref):
    d = df_ref[...]
    i = if_ref[...]
    t = tmpl_ref[...]
    rmin = jnp.min(d, axis=1, keepdims=True)
    cand = jnp.where(d == rmin, i, BIG_I)
    imin = jnp.min(cand, axis=1, keepdims=True)
    tt = t[:, 0:1] * t[:, 0:1] + t[:, 1:2] * t[:, 1:2] + t[:, 2:3] * t[:, 2:3]
    outd_ref[...] = rmin + tt
    outi_ref[...] = imin


def _reduce_call(dfold, ifold, tmpl):
    return pl.pallas_call(
        _reduce_body,
        grid=(M_TMPL // RM2,),
        in_specs=[
            pl.BlockSpec((RM2, 128), lambda i: (i, 0)),
            pl.BlockSpec((RM2, 128), lambda i: (i, 0)),
            pl.BlockSpec((RM2, 3), lambda i: (i, 0)),
        ],
        out_specs=[
            pl.BlockSpec((RM2, 1), lambda i: (i, 0)),
            pl.BlockSpec((RM2, 1), lambda i: (i, 0)),
        ],
        out_shape=[
            jax.ShapeDtypeStruct((M_TMPL, 1), jnp.float32),
            jax.ShapeDtypeStruct((M_TMPL, 1), jnp.int32),
        ],
        compiler_params=pltpu.CompilerParams(
            dimension_semantics=("parallel",)),
    )(dfold, ifold, tmpl)


def _sc_gather_finish(dists, idx, snx, sny, snz, tnx, tny, tnz):
    mesh = plsc.VectorSubcoreMesh(
        core_axis_name="c", subcore_axis_name="s",
        num_cores=NUM_CORES, num_subcores=NUM_SUBCORES)

    @functools.partial(
        pl.kernel,
        out_type=jax.ShapeDtypeStruct((NUM_TILES, LANES), jnp.float32),
        mesh=mesh,
        scratch_types=[
            pltpu.VMEM((PER_TILE,), jnp.int32),
            pltpu.VMEM((PER_TILE,), jnp.float32),
            pltpu.VMEM((PER_TILE,), jnp.float32),
            pltpu.VMEM((PER_TILE,), jnp.float32),
            pltpu.VMEM((PER_TILE,), jnp.float32),
            pltpu.VMEM((PER_TILE,), jnp.float32),
            pltpu.VMEM((PER_TILE,), jnp.float32),
            pltpu.VMEM((PER_TILE,), jnp.float32),
            pltpu.VMEM((LANES,), jnp.float32),
            pltpu.SemaphoreType.DMA,
        ],
    )
    def sck(d_hbm, i_hbm, snx_hbm, sny_hbm, snz_hbm,
            tnx_hbm, tny_hbm, tnz_hbm, out_hbm,
            idx_v, d_v, gx_v, gy_v, gz_v, tx_v, ty_v, tz_v, acc_v, sem):
        wid = lax.axis_index("s") * NUM_CORES + lax.axis_index("c")
        base = wid * PER_TILE
        pltpu.sync_copy(i_hbm.at[pl.ds(base, PER_TILE)], idx_v)
        pltpu.sync_copy(d_hbm.at[pl.ds(base, PER_TILE)], d_v)
        pltpu.sync_copy(tnx_hbm.at[pl.ds(base, PER_TILE)], tx_v)
        pltpu.sync_copy(tny_hbm.at[pl.ds(base, PER_TILE)], ty_v)
        pltpu.sync_copy(tnz_hbm.at[pl.ds(base, PER_TILE)], tz_v)
        pltpu.async_copy(snx_hbm.at[idx_v], gx_v, sem).wait()
        pltpu.async_copy(sny_hbm.at[idx_v], gy_v, sem).wait()
        pltpu.async_copy(snz_hbm.at[idx_v], gz_v, sem).wait()
        acc = jnp.zeros((LANES,), jnp.float32)
        for j in range(PER_TILE // LANES):
            sl = pl.ds(j * LANES, LANES)
            dot = gx_v[sl] * tx_v[sl] + gy_v[sl] * ty_v[sl] + gz_v[sl] * tz_v[sl]
            acc = acc + jnp.where(dot > 0.5, d_v[sl], 0.0)
        acc_v[...] = acc
        pltpu.sync_copy(acc_v, out_hbm.at[wid])

    return sck(dists, idx, snx, sny, snz, tnx, tny, tnz)


def kernel(scan_vertices, template_vertices, scan_normals, template_normals):
    scan3 = jnp.pad(scan_vertices.T, ((0, 0), (0, NPAD - N_SCAN)),
                    constant_values=PAD_VAL).reshape(3, NROWS, 128)
    ssq = _ssq_call(scan3)
    dfold, ifold = _nn_call(template_vertices, scan3, ssq)
    d2, i2 = _reduce_call(dfold, ifold, template_vertices)
    dists = d2[:, 0]
    idx = i2[:, 0]
    snx, sny, snz = scan_normals[:, 0], scan_normals[:, 1], scan_normals[:, 2]
    tnx, tny, tnz = (template_normals[:, 0], template_normals[:, 1],
                     template_normals[:, 2])
    partials = _sc_gather_finish(dists, idx, snx, sny, snz, tnx, tny, tnz)
    return jnp.sum(partials)


# speedup vs baseline: 1.3847x; 1.2552x over previous
import functools

import jax
import jax.numpy as jnp
from jax import lax
from jax.experimental import pallas as pl
from jax.experimental.pallas import tpu as pltpu
from jax.experimental.pallas import tpu_sc as plsc

N_SCAN = 50000
M_TMPL = 8192
NPAD = 51200
NROWS = NPAD // 128
NBLK = NROWS // 8
BM = 16
PAD_VAL = 1e18
BIG_I = 2147483647
RM2 = 512

NUM_CORES = 2
NUM_SUBCORES = 16
NUM_TILES = NUM_CORES * NUM_SUBCORES
PER_TILE = M_TMPL // NUM_TILES
LANES = 16


def _ssq_body(scan_ref, out_ref):
    s0 = scan_ref[0]
    s1 = scan_ref[1]
    s2 = scan_ref[2]
    out_ref[...] = s0 * s0 + s1 * s1 + s2 * s2


def _ssq_call(scan3):
    return pl.pallas_call(
        _ssq_body,
        out_shape=jax.ShapeDtypeStruct((NROWS, 128), jnp.float32),
    )(scan3)


def _nn_body(tmpl_ref, scan_ref, ssq_ref, outd_ref, outi_ref):
    a0 = [tmpl_ref[m, 0] * -2.0 for m in range(BM)]
    a1 = [tmpl_ref[m, 1] * -2.0 for m in range(BM)]
    a2 = [tmpl_ref[m, 2] * -2.0 for m in range(BM)]

    bds = [jnp.full((8, 128), jnp.inf, jnp.float32) for _ in range(BM)]
    bis = [jnp.zeros((8, 128), jnp.int32) for _ in range(BM)]
    for b in range(NBLK):
        base = 8 * b
        s0 = scan_ref[0, pl.ds(base, 8), :]
        s1 = scan_ref[1, pl.ds(base, 8), :]
        s2 = scan_ref[2, pl.ds(base, 8), :]
        sq = ssq_ref[pl.ds(base, 8), :]
        for m in range(BM):
            d = sq + a0[m] * s0
            d = d + a1[m] * s1
            d = d + a2[m] * s2
            upd = d < bds[m]
            bis[m] = jnp.where(upd, b, bis[m])
            bds[m] = jnp.minimum(d, bds[m])

    sub_l = lax.broadcasted_iota(jnp.int32, (8, 128), 0) * 128
    lane = lax.broadcasted_iota(jnp.int32, (8, 128), 1)
    rows_d, rows_i = [], []
    for m in range(BM):
        bd, bi = bds[m], bis[m]
        n_idx = bi * 1024 + sub_l + lane
        dmin = jnp.min(bd, axis=0, keepdims=True)
        cand = jnp.where(bd == dmin, n_idx, BIG_I)
        rows_d.append(dmin)
        rows_i.append(jnp.min(cand, axis=0, keepdims=True))
    outd_ref[...] = jnp.concatenate(rows_d, axis=0)
    outi_ref[...] = jnp.concatenate(rows_i, axis=0)


def _nn_call(tmpl, scan3, ssq):
    return pl.pallas_call(
        _nn_body,
        grid=(M_TMPL // BM,),
        in_specs=[
            pl.BlockSpec((BM, 3), lambda i: (i, 0),
                         memory_space=pltpu.SMEM),
            pl.BlockSpec((3, NROWS, 128), lambda i: (0, 0, 0)),
            pl.BlockSpec((NROWS, 128), lambda i: (0, 0)),
        ],
        out_specs=[
            pl.BlockSpec((BM, 128), lambda i: (i, 0)),
            pl.BlockSpec((BM, 128), lambda i: (i, 0)),
        ],
        out_shape=[
            jax.ShapeDtypeStruct((M_TMPL, 128), jnp.float32),
            jax.ShapeDtypeStruct((M_TMPL, 128), jnp.int32),
        ],
        compiler_params=pltpu.CompilerParams(
            dimension_semantics=("parallel",)),
    )(tmpl, scan3, ssq)


def _reduce_body(df_ref, if_ref, tmpl_ref, outd_ref, outi_ref):
    d = df_ref[...]
    i = if_ref[...]
    t = tmpl_ref[...]
    rmin = jnp.min(d, axis=1, keepdims=True)
    cand = jnp.where(d == rmin, i, BIG_I)
    imin = jnp.min(cand, axis=1, keepdims=True)
    tt = t[:, 0:1] * t[:, 0:1] + t[:, 1:2] * t[:, 1:2] + t[:, 2:3] * t[:, 2:3]
    outd_ref[...] = rmin + tt
    outi_ref[...] = imin


def _reduce_call(dfold, ifold, tmpl):
    return pl.pallas_call(
        _reduce_body,
        grid=(M_TMPL // RM2,),
        in_specs=[
            pl.BlockSpec((RM2, 128), lambda i: (i, 0)),
            pl.BlockSpec((RM2, 128), lambda i: (i, 0)),
            pl.BlockSpec((RM2, 3), lambda i: (i, 0)),
        ],
        out_specs=[
            pl.BlockSpec((RM2, 1), lambda i: (i, 0)),
            pl.BlockSpec((RM2, 1), lambda i: (i, 0)),
        ],
        out_shape=[
            jax.ShapeDtypeStruct((M_TMPL, 1), jnp.float32),
            jax.ShapeDtypeStruct((M_TMPL, 1), jnp.int32),
        ],
        compiler_params=pltpu.CompilerParams(
            dimension_semantics=("parallel",)),
    )(dfold, ifold, tmpl)


def _sc_gather_finish(dists, idx, snx, sny, snz, tnx, tny, tnz):
    mesh = plsc.VectorSubcoreMesh(
        core_axis_name="c", subcore_axis_name="s",
        num_cores=NUM_CORES, num_subcores=NUM_SUBCORES)

    @functools.partial(
        pl.kernel,
        out_type=jax.ShapeDtypeStruct((NUM_TILES, LANES), jnp.float32),
        mesh=mesh,
        scratch_types=[
            pltpu.VMEM((PER_TILE,), jnp.int32),
            pltpu.VMEM((PER_TILE,), jnp.float32),
            pltpu.VMEM((PER_TILE,), jnp.float32),
            pltpu.VMEM((PER_TILE,), jnp.float32),
            pltpu.VMEM((PER_TILE,), jnp.float32),
            pltpu.VMEM((PER_TILE,), jnp.float32),
            pltpu.VMEM((PER_TILE,), jnp.float32),
            pltpu.VMEM((PER_TILE,), jnp.float32),
            pltpu.VMEM((LANES,), jnp.float32),
            pltpu.SemaphoreType.DMA,
        ],
    )
    def sck(d_hbm, i_hbm, snx_hbm, sny_hbm, snz_hbm,
            tnx_hbm, tny_hbm, tnz_hbm, out_hbm,
            idx_v, d_v, gx_v, gy_v, gz_v, tx_v, ty_v, tz_v, acc_v, sem):
        wid = lax.axis_index("s") * NUM_CORES + lax.axis_index("c")
        base = wid * PER_TILE
        pltpu.sync_copy(i_hbm.at[pl.ds(base, PER_TILE)], idx_v)
        pltpu.sync_copy(d_hbm.at[pl.ds(base, PER_TILE)], d_v)
        pltpu.sync_copy(tnx_hbm.at[pl.ds(base, PER_TILE)], tx_v)
        pltpu.sync_copy(tny_hbm.at[pl.ds(base, PER_TILE)], ty_v)
        pltpu.sync_copy(tnz_hbm.at[pl.ds(base, PER_TILE)], tz_v)
        pltpu.async_copy(snx_hbm.at[idx_v], gx_v, sem).wait()
        pltpu.async_copy(sny_hbm.at[idx_v], gy_v, sem).wait()
        pltpu.async_copy(snz_hbm.at[idx_v], gz_v, sem).wait()
        acc = jnp.zeros((LANES,), jnp.float32)
        for j in range(PER_TILE // LANES):
            sl = pl.ds(j * LANES, LANES)
            dot = gx_v[sl] * tx_v[sl] + gy_v[sl] * ty_v[sl] + gz_v[sl] * tz_v[sl]
            acc = acc + jnp.where(dot > 0.5, d_v[sl], 0.0)
        acc_v[...] = acc
        pltpu.sync_copy(acc_v, out_hbm.at[wid])

    return sck(dists, idx, snx, sny, snz, tnx, tny, tnz)


def kernel(scan_vertices, template_vertices, scan_normals, template_normals):
    scan3 = jnp.pad(scan_vertices.T, ((0, 0), (0, NPAD - N_SCAN)),
                    constant_values=PAD_VAL).reshape(3, NROWS, 128)
    ssq = _ssq_call(scan3)
    dfold, ifold = _nn_call(template_vertices, scan3, ssq)
    d2, i2 = _reduce_call(dfold, ifold, template_vertices)
    dists = d2[:, 0]
    idx = i2[:, 0]
    snx, sny, snz = scan_normals[:, 0], scan_normals[:, 1], scan_normals[:, 2]
    tnx, tny, tnz = (template_normals[:, 0], template_normals[:, 1],
                     template_normals[:, 2])
    partials = _sc_gather_finish(dists, idx, snx, sny, snz, tnx, tny, tnz)
    return jnp.sum(partials)


# speedup vs baseline: 1.3857x; 1.0007x over previous
import functools

import jax
import jax.numpy as jnp
from jax import lax
from jax.experimental import pallas as pl
from jax.experimental.pallas import tpu as pltpu
from jax.experimental.pallas import tpu_sc as plsc

N_SCAN = 50000
M_TMPL = 8192
NPAD = 51200
NROWS = NPAD // 128
NBLK = NROWS // 8
BM = 16
PAD_VAL = 1e18
BIG_I = 2147483647
RM2 = 512

NUM_CORES = 2
NUM_SUBCORES = 16
NUM_TILES = NUM_CORES * NUM_SUBCORES
PER_TILE = M_TMPL // NUM_TILES
LANES = 16


def _ssq_body(scan_ref, out_ref):
    s0 = scan_ref[0]
    s1 = scan_ref[1]
    s2 = scan_ref[2]
    out_ref[...] = s0 * s0 + s1 * s1 + s2 * s2


def _ssq_call(scan3):
    return pl.pallas_call(
        _ssq_body,
        out_shape=jax.ShapeDtypeStruct((NROWS, 128), jnp.float32),
    )(scan3)


def _nn_body(tmpl_ref, scan_ref, ssq_ref, outd_ref, outi_ref):
    base_m = pl.program_id(0) * (3 * BM)
    a0 = [tmpl_ref[base_m + 3 * m] for m in range(BM)]
    a1 = [tmpl_ref[base_m + 3 * m + 1] for m in range(BM)]
    a2 = [tmpl_ref[base_m + 3 * m + 2] for m in range(BM)]

    bds = [jnp.full((8, 128), jnp.inf, jnp.float32) for _ in range(BM)]
    bis = [jnp.zeros((8, 128), jnp.int32) for _ in range(BM)]
    for b in range(NBLK):
        base = 8 * b
        s0 = scan_ref[0, pl.ds(base, 8), :]
        s1 = scan_ref[1, pl.ds(base, 8), :]
        s2 = scan_ref[2, pl.ds(base, 8), :]
        sq = ssq_ref[pl.ds(base, 8), :]
        for m in range(BM):
            d = sq + a0[m] * s0
            d = d + a1[m] * s1
            d = d + a2[m] * s2
            upd = d < bds[m]
            bis[m] = jnp.where(upd, b, bis[m])
            bds[m] = jnp.minimum(d, bds[m])

    sub_l = lax.broadcasted_iota(jnp.int32, (8, 128), 0) * 128
    lane = lax.broadcasted_iota(jnp.int32, (8, 128), 1)
    rows_d, rows_i = [], []
    for m in range(BM):
        bd, bi = bds[m], bis[m]
        n_idx = bi * 1024 + sub_l + lane
        dmin = jnp.min(bd, axis=0, keepdims=True)
        cand = jnp.where(bd == dmin, n_idx, BIG_I)
        rows_d.append(dmin)
        rows_i.append(jnp.min(cand, axis=0, keepdims=True))
    outd_ref[...] = jnp.concatenate(rows_d, axis=0)
    outi_ref[...] = jnp.concatenate(rows_i, axis=0)


def _nn_call(tmpl, scan3, ssq):
    return pl.pallas_call(
        _nn_body,
        grid=(M_TMPL // BM,),
        in_specs=[
            pl.BlockSpec((M_TMPL * 3,), lambda i: (0,),
                         memory_space=pltpu.SMEM),
            pl.BlockSpec((3, NROWS, 128), lambda i: (0, 0, 0)),
            pl.BlockSpec((NROWS, 128), lambda i: (0, 0)),
        ],
        out_specs=[
            pl.BlockSpec((BM, 128), lambda i: (i, 0)),
            pl.BlockSpec((BM, 128), lambda i: (i, 0)),
        ],
        out_shape=[
            jax.ShapeDtypeStruct((M_TMPL, 128), jnp.float32),
            jax.ShapeDtypeStruct((M_TMPL, 128), jnp.int32),
        ],
        compiler_params=pltpu.CompilerParams(
            dimension_semantics=("parallel",)),
    )(tmpl, scan3, ssq)


def _reduce_body(df_ref, if_ref, tmpl_ref, outd_ref, outi_ref):
    d = df_ref[...]
    i = if_ref[...]
    t = tmpl_ref[...]
    rmin = jnp.min(d, axis=1, keepdims=True)
    cand = jnp.where(d == rmin, i, BIG_I)
    imin = jnp.min(cand, axis=1, keepdims=True)
    tt = t[:, 0:1] * t[:, 0:1] + t[:, 1:2] * t[:, 1:2] + t[:, 2:3] * t[:, 2:3]
    outd_ref[...] = rmin + tt
    outi_ref[...] = imin


def _reduce_call(dfold, ifold, tmpl):
    return pl.pallas_call(
        _reduce_body,
        grid=(M_TMPL // RM2,),
        in_specs=[
            pl.BlockSpec((RM2, 128), lambda i: (i, 0)),
            pl.BlockSpec((RM2, 128), lambda i: (i, 0)),
            pl.BlockSpec((RM2, 3), lambda i: (i, 0)),
        ],
        out_specs=[
            pl.BlockSpec((RM2, 1), lambda i: (i, 0)),
            pl.BlockSpec((RM2, 1), lambda i: (i, 0)),
        ],
        out_shape=[
            jax.ShapeDtypeStruct((M_TMPL, 1), jnp.float32),
            jax.ShapeDtypeStruct((M_TMPL, 1), jnp.int32),
        ],
        compiler_params=pltpu.CompilerParams(
            dimension_semantics=("parallel",)),
    )(dfold, ifold, tmpl)


def _sc_gather_finish(dists, idx, snx, sny, snz, tnx, tny, tnz):
    mesh = plsc.VectorSubcoreMesh(
        core_axis_name="c", subcore_axis_name="s",
        num_cores=NUM_CORES, num_subcores=NUM_SUBCORES)

    @functools.partial(
        pl.kernel,
        out_type=jax.ShapeDtypeStruct((NUM_TILES, LANES), jnp.float32),
        mesh=mesh,
        scratch_types=[
            pltpu.VMEM((PER_TILE,), jnp.int32),
            pltpu.VMEM((PER_TILE,), jnp.float32),
            pltpu.VMEM((PER_TILE,), jnp.float32),
            pltpu.VMEM((PER_TILE,), jnp.float32),
            pltpu.VMEM((PER_TILE,), jnp.float32),
            pltpu.VMEM((PER_TILE,), jnp.float32),
            pltpu.VMEM((PER_TILE,), jnp.float32),
            pltpu.VMEM((PER_TILE,), jnp.float32),
            pltpu.VMEM((LANES,), jnp.float32),
            pltpu.SemaphoreType.DMA,
        ],
    )
    def sck(d_hbm, i_hbm, snx_hbm, sny_hbm, snz_hbm,
            tnx_hbm, tny_hbm, tnz_hbm, out_hbm,
            idx_v, d_v, gx_v, gy_v, gz_v, tx_v, ty_v, tz_v, acc_v, sem):
        wid = lax.axis_index("s") * NUM_CORES + lax.axis_index("c")
        base = wid * PER_TILE
        pltpu.sync_copy(i_hbm.at[pl.ds(base, PER_TILE)], idx_v)
        pltpu.sync_copy(d_hbm.at[pl.ds(base, PER_TILE)], d_v)
        pltpu.sync_copy(tnx_hbm.at[pl.ds(base, PER_TILE)], tx_v)
        pltpu.sync_copy(tny_hbm.at[pl.ds(base, PER_TILE)], ty_v)
        pltpu.sync_copy(tnz_hbm.at[pl.ds(base, PER_TILE)], tz_v)
        pltpu.async_copy(snx_hbm.at[idx_v], gx_v, sem).wait()
        pltpu.async_copy(sny_hbm.at[idx_v], gy_v, sem).wait()
        pltpu.async_copy(snz_hbm.at[idx_v], gz_v, sem).wait()
        acc = jnp.zeros((LANES,), jnp.float32)
        for j in range(PER_TILE // LANES):
            sl = pl.ds(j * LANES, LANES)
            dot = gx_v[sl] * tx_v[sl] + gy_v[sl] * ty_v[sl] + gz_v[sl] * tz_v[sl]
            acc = acc + jnp.where(dot > 0.5, d_v[sl], 0.0)
        acc_v[...] = acc
        pltpu.sync_copy(acc_v, out_hbm.at[wid])

    return sck(dists, idx, snx, sny, snz, tnx, tny, tnz)


def kernel(scan_vertices, template_vertices, scan_normals, template_normals):
    scan3 = jnp.pad(scan_vertices.T, ((0, 0), (0, NPAD - N_SCAN)),
                    constant_values=PAD_VAL).reshape(3, NROWS, 128)
    ssq = _ssq_call(scan3)
    dfold, ifold = _nn_call((template_vertices * -2.0).reshape(-1), scan3, ssq)
    d2, i2 = _reduce_call(dfold, ifold, template_vertices)
    dists = d2[:, 0]
    idx = i2[:, 0]
    snx, sny, snz = scan_normals[:, 0], scan_normals[:, 1], scan_normals[:, 2]
    tnx, tny, tnz = (template_normals[:, 0], template_normals[:, 1],
                     template_normals[:, 2])
    partials = _sc_gather_finish(dists, idx, snx, sny, snz, tnx, tny, tnz)
    return jnp.sum(partials)


# speedup vs baseline: 1.4096x; 1.0172x over previous
import functools

import jax
import jax.numpy as jnp
from jax import lax
from jax.experimental import pallas as pl
from jax.experimental.pallas import tpu as pltpu
from jax.experimental.pallas import tpu_sc as plsc

N_SCAN = 50000
M_TMPL = 8192
NPAD = 51200
NROWS = NPAD // 128
NBLK = NROWS // 8
BM = 16
NGRP = 2
PAD_VAL = 1e18
BIG_I = 2147483647
RM2 = 512

NUM_CORES = 2
NUM_SUBCORES = 16
NUM_TILES = NUM_CORES * NUM_SUBCORES
PER_TILE = M_TMPL // NUM_TILES
LANES = 16


def _ssq_body(scan_ref, out_ref):
    s0 = scan_ref[0]
    s1 = scan_ref[1]
    s2 = scan_ref[2]
    out_ref[...] = s0 * s0 + s1 * s1 + s2 * s2


def _ssq_call(scan3):
    return pl.pallas_call(
        _ssq_body,
        out_shape=jax.ShapeDtypeStruct((NROWS, 128), jnp.float32),
    )(scan3)


def _nn_body(tmpl_ref, scan_ref, ssq_ref, outd_ref, outi_ref):
    sub_l = lax.broadcasted_iota(jnp.int32, (8, 128), 0) * 128
    lane = lax.broadcasted_iota(jnp.int32, (8, 128), 1)
    for g in range(NGRP):
        base_m = (pl.program_id(0) * NGRP + g) * (3 * BM)
        a0 = [tmpl_ref[base_m + 3 * m] for m in range(BM)]
        a1 = [tmpl_ref[base_m + 3 * m + 1] for m in range(BM)]
        a2 = [tmpl_ref[base_m + 3 * m + 2] for m in range(BM)]

        bds = [jnp.full((8, 128), jnp.inf, jnp.float32) for _ in range(BM)]
        bis = [jnp.zeros((8, 128), jnp.int32) for _ in range(BM)]
        for b in range(NBLK):
            base = 8 * b
            s0 = scan_ref[0, pl.ds(base, 8), :]
            s1 = scan_ref[1, pl.ds(base, 8), :]
            s2 = scan_ref[2, pl.ds(base, 8), :]
            sq = ssq_ref[pl.ds(base, 8), :]
            for m in range(BM):
                d = sq + a0[m] * s0
                d = d + a1[m] * s1
                d = d + a2[m] * s2
                upd = d < bds[m]
                bis[m] = jnp.where(upd, b, bis[m])
                bds[m] = jnp.minimum(d, bds[m])

        rows_d, rows_i = [], []
        for m in range(BM):
            bd, bi = bds[m], bis[m]
            n_idx = bi * 1024 + sub_l + lane
            dmin = jnp.min(bd, axis=0, keepdims=True)
            cand = jnp.where(bd == dmin, n_idx, BIG_I)
            rows_d.append(dmin)
            rows_i.append(jnp.min(cand, axis=0, keepdims=True))
        outd_ref[pl.ds(g * BM, BM), :] = jnp.concatenate(rows_d, axis=0)
        outi_ref[pl.ds(g * BM, BM), :] = jnp.concatenate(rows_i, axis=0)


def _nn_call(tmpl, scan3, ssq):
    return pl.pallas_call(
        _nn_body,
        grid=(M_TMPL // (BM * NGRP),),
        in_specs=[
            pl.BlockSpec((M_TMPL * 3,), lambda i: (0,),
                         memory_space=pltpu.SMEM),
            pl.BlockSpec((3, NROWS, 128), lambda i: (0, 0, 0)),
            pl.BlockSpec((NROWS, 128), lambda i: (0, 0)),
        ],
        out_specs=[
            pl.BlockSpec((BM * NGRP, 128), lambda i: (i, 0)),
            pl.BlockSpec((BM * NGRP, 128), lambda i: (i, 0)),
        ],
        out_shape=[
            jax.ShapeDtypeStruct((M_TMPL, 128), jnp.float32),
            jax.ShapeDtypeStruct((M_TMPL, 128), jnp.int32),
        ],
        compiler_params=pltpu.CompilerParams(
            dimension_semantics=("parallel",)),
    )(tmpl, scan3, ssq)


def _reduce_body(df_ref, if_ref, tmpl_ref, outd_ref, outi_ref):
    d = df_ref[...]
    i = if_ref[...]
    t = tmpl_ref[...]
    rmin = jnp.min(d, axis=1, keepdims=True)
    cand = jnp.where(d == rmin, i, BIG_I)
    imin = jnp.min(cand, axis=1, keepdims=True)
    tt = t[:, 0:1] * t[:, 0:1] + t[:, 1:2] * t[:, 1:2] + t[:, 2:3] * t[:, 2:3]
    outd_ref[...] = rmin + tt
    outi_ref[...] = imin


def _reduce_call(dfold, ifold, tmpl):
    return pl.pallas_call(
        _reduce_body,
        grid=(M_TMPL // RM2,),
        in_specs=[
            pl.BlockSpec((RM2, 128), lambda i: (i, 0)),
            pl.BlockSpec((RM2, 128), lambda i: (i, 0)),
            pl.BlockSpec((RM2, 3), lambda i: (i, 0)),
        ],
        out_specs=[
            pl.BlockSpec((RM2, 1), lambda i: (i, 0)),
            pl.BlockSpec((RM2, 1), lambda i: (i, 0)),
        ],
        out_shape=[
            jax.ShapeDtypeStruct((M_TMPL, 1), jnp.float32),
            jax.ShapeDtypeStruct((M_TMPL, 1), jnp.int32),
        ],
        compiler_params=pltpu.CompilerParams(
            dimension_semantics=("parallel",)),
    )(dfold, ifold, tmpl)


def _sc_gather_finish(dists, idx, snx, sny, snz, tnx, tny, tnz):
    mesh = plsc.VectorSubcoreMesh(
        core_axis_name="c", subcore_axis_name="s",
        num_cores=NUM_CORES, num_subcores=NUM_SUBCORES)

    @functools.partial(
        pl.kernel,
        out_type=jax.ShapeDtypeStruct((NUM_TILES, LANES), jnp.float32),
        mesh=mesh,
        scratch_types=[
            pltpu.VMEM((PER_TILE,), jnp.int32),
            pltpu.VMEM((PER_TILE,), jnp.float32),
            pltpu.VMEM((PER_TILE,), jnp.float32),
            pltpu.VMEM((PER_TILE,), jnp.float32),
            pltpu.VMEM((PER_TILE,), jnp.float32),
            pltpu.VMEM((PER_TILE,), jnp.float32),
            pltpu.VMEM((PER_TILE,), jnp.float32),
            pltpu.VMEM((PER_TILE,), jnp.float32),
            pltpu.VMEM((LANES,), jnp.float32),
            pltpu.SemaphoreType.DMA,
        ],
    )
    def sck(d_hbm, i_hbm, snx_hbm, sny_hbm, snz_hbm,
            tnx_hbm, tny_hbm, tnz_hbm, out_hbm,
            idx_v, d_v, gx_v, gy_v, gz_v, tx_v, ty_v, tz_v, acc_v, sem):
        wid = lax.axis_index("s") * NUM_CORES + lax.axis_index("c")
        base = wid * PER_TILE
        pltpu.sync_copy(i_hbm.at[pl.ds(base, PER_TILE)], idx_v)
        pltpu.sync_copy(d_hbm.at[pl.ds(base, PER_TILE)], d_v)
        pltpu.sync_copy(tnx_hbm.at[pl.ds(base, PER_TILE)], tx_v)
        pltpu.sync_copy(tny_hbm.at[pl.ds(base, PER_TILE)], ty_v)
        pltpu.sync_copy(tnz_hbm.at[pl.ds(base, PER_TILE)], tz_v)
        pltpu.async_copy(snx_hbm.at[idx_v], gx_v, sem).wait()
        pltpu.async_copy(sny_hbm.at[idx_v], gy_v, sem).wait()
        pltpu.async_copy(snz_hbm.at[idx_v], gz_v, sem).wait()
        acc = jnp.zeros((LANES,), jnp.float32)
        for j in range(PER_TILE // LANES):
            sl = pl.ds(j * LANES, LANES)
            dot = gx_v[sl] * tx_v[sl] + gy_v[sl] * ty_v[sl] + gz_v[sl] * tz_v[sl]
            acc = acc + jnp.where(dot > 0.5, d_v[sl], 0.0)
        acc_v[...] = acc
        pltpu.sync_copy(acc_v, out_hbm.at[wid])

    return sck(dists, idx, snx, sny, snz, tnx, tny, tnz)


def kernel(scan_vertices, template_vertices, scan_normals, template_normals):
    scan3 = jnp.pad(scan_vertices.T, ((0, 0), (0, NPAD - N_SCAN)),
                    constant_values=PAD_VAL).reshape(3, NROWS, 128)
    ssq = _ssq_call(scan3)
    dfold, ifold = _nn_call((template_vertices * -2.0).reshape(-1), scan3, ssq)
    d2, i2 = _reduce_call(dfold, ifold, template_vertices)
    dists = d2[:, 0]
    idx = i2[:, 0]
    snx, sny, snz = scan_normals[:, 0], scan_normals[:, 1], scan_normals[:, 2]
    tnx, tny, tnz = (template_normals[:, 0], template_normals[:, 1],
                     template_normals[:, 2])
    partials = _sc_gather_finish(dists, idx, snx, sny, snz, tnx, tny, tnz)
    return jnp.sum(partials)


# speedup vs baseline: 1.4206x; 1.0078x over previous
import functools

import jax
import jax.numpy as jnp
from jax import lax
from jax.experimental import pallas as pl
from jax.experimental.pallas import tpu as pltpu
from jax.experimental.pallas import tpu_sc as plsc

N_SCAN = 50000
M_TMPL = 8192
NPAD = 51200
NROWS = NPAD // 128
NBLK = NROWS // 8
BM = 16
NGRP = 4
PAD_VAL = 1e18
BIG_I = 2147483647
RM2 = 512

NUM_CORES = 2
NUM_SUBCORES = 16
NUM_TILES = NUM_CORES * NUM_SUBCORES
PER_TILE = M_TMPL // NUM_TILES
LANES = 16


def _ssq_body(scan_ref, out_ref):
    s0 = scan_ref[0]
    s1 = scan_ref[1]
    s2 = scan_ref[2]
    out_ref[...] = s0 * s0 + s1 * s1 + s2 * s2


def _ssq_call(scan3):
    return pl.pallas_call(
        _ssq_body,
        out_shape=jax.ShapeDtypeStruct((NROWS, 128), jnp.float32),
    )(scan3)


def _nn_body(tmpl_ref, scan_ref, ssq_ref, outd_ref, outi_ref):
    sub_l = lax.broadcasted_iota(jnp.int32, (8, 128), 0) * 128
    lane = lax.broadcasted_iota(jnp.int32, (8, 128), 1)
    for g in range(NGRP):
        base_m = (pl.program_id(0) * NGRP + g) * (3 * BM)
        a0 = [tmpl_ref[base_m + 3 * m] for m in range(BM)]
        a1 = [tmpl_ref[base_m + 3 * m + 1] for m in range(BM)]
        a2 = [tmpl_ref[base_m + 3 * m + 2] for m in range(BM)]

        bds = [jnp.full((8, 128), jnp.inf, jnp.float32) for _ in range(BM)]
        bis = [jnp.zeros((8, 128), jnp.int32) for _ in range(BM)]
        for b in range(NBLK):
            base = 8 * b
            s0 = scan_ref[0, pl.ds(base, 8), :]
            s1 = scan_ref[1, pl.ds(base, 8), :]
            s2 = scan_ref[2, pl.ds(base, 8), :]
            sq = ssq_ref[pl.ds(base, 8), :]
            for m in range(BM):
                d = sq + a0[m] * s0
                d = d + a1[m] * s1
                d = d + a2[m] * s2
                upd = d < bds[m]
                bis[m] = jnp.where(upd, b, bis[m])
                bds[m] = jnp.minimum(d, bds[m])

        rows_d, rows_i = [], []
        for m in range(BM):
            bd, bi = bds[m], bis[m]
            n_idx = bi * 1024 + sub_l + lane
            dmin = jnp.min(bd, axis=0, keepdims=True)
            cand = jnp.where(bd == dmin, n_idx, BIG_I)
            rows_d.append(dmin)
            rows_i.append(jnp.min(cand, axis=0, keepdims=True))
        outd_ref[pl.ds(g * BM, BM), :] = jnp.concatenate(rows_d, axis=0)
        outi_ref[pl.ds(g * BM, BM), :] = jnp.concatenate(rows_i, axis=0)


def _nn_call(tmpl, scan3, ssq):
    return pl.pallas_call(
        _nn_body,
        grid=(M_TMPL // (BM * NGRP),),
        in_specs=[
            pl.BlockSpec((M_TMPL * 3,), lambda i: (0,),
                         memory_space=pltpu.SMEM),
            pl.BlockSpec((3, NROWS, 128), lambda i: (0, 0, 0)),
            pl.BlockSpec((NROWS, 128), lambda i: (0, 0)),
        ],
        out_specs=[
            pl.BlockSpec((BM * NGRP, 128), lambda i: (i, 0)),
            pl.BlockSpec((BM * NGRP, 128), lambda i: (i, 0)),
        ],
        out_shape=[
            jax.ShapeDtypeStruct((M_TMPL, 128), jnp.float32),
            jax.ShapeDtypeStruct((M_TMPL, 128), jnp.int32),
        ],
        compiler_params=pltpu.CompilerParams(
            dimension_semantics=("parallel",)),
    )(tmpl, scan3, ssq)


def _reduce_body(df_ref, if_ref, tmpl_ref, outd_ref, outi_ref):
    d = df_ref[...]
    i = if_ref[...]
    t = tmpl_ref[...]
    rmin = jnp.min(d, axis=1, keepdims=True)
    cand = jnp.where(d == rmin, i, BIG_I)
    imin = jnp.min(cand, axis=1, keepdims=True)
    tt = t[:, 0:1] * t[:, 0:1] + t[:, 1:2] * t[:, 1:2] + t[:, 2:3] * t[:, 2:3]
    outd_ref[...] = rmin + tt
    outi_ref[...] = imin


def _reduce_call(dfold, ifold, tmpl):
    return pl.pallas_call(
        _reduce_body,
        grid=(M_TMPL // RM2,),
        in_specs=[
            pl.BlockSpec((RM2, 128), lambda i: (i, 0)),
            pl.BlockSpec((RM2, 128), lambda i: (i, 0)),
            pl.BlockSpec((RM2, 3), lambda i: (i, 0)),
        ],
        out_specs=[
            pl.BlockSpec((RM2, 1), lambda i: (i, 0)),
            pl.BlockSpec((RM2, 1), lambda i: (i, 0)),
        ],
        out_shape=[
            jax.ShapeDtypeStruct((M_TMPL, 1), jnp.float32),
            jax.ShapeDtypeStruct((M_TMPL, 1), jnp.int32),
        ],
        compiler_params=pltpu.CompilerParams(
            dimension_semantics=("parallel",)),
    )(dfold, ifold, tmpl)


def _sc_gather_finish(dists, idx, snx, sny, snz, tnx, tny, tnz):
    mesh = plsc.VectorSubcoreMesh(
        core_axis_name="c", subcore_axis_name="s",
        num_cores=NUM_CORES, num_subcores=NUM_SUBCORES)

    @functools.partial(
        pl.kernel,
        out_type=jax.ShapeDtypeStruct((NUM_TILES, LANES), jnp.float32),
        mesh=mesh,
        scratch_types=[
            pltpu.VMEM((PER_TILE,), jnp.int32),
            pltpu.VMEM((PER_TILE,), jnp.float32),
            pltpu.VMEM((PER_TILE,), jnp.float32),
            pltpu.VMEM((PER_TILE,), jnp.float32),
            pltpu.VMEM((PER_TILE,), jnp.float32),
            pltpu.VMEM((PER_TILE,), jnp.float32),
            pltpu.VMEM((PER_TILE,), jnp.float32),
            pltpu.VMEM((PER_TILE,), jnp.float32),
            pltpu.VMEM((LANES,), jnp.float32),
            pltpu.SemaphoreType.DMA,
        ],
    )
    def sck(d_hbm, i_hbm, snx_hbm, sny_hbm, snz_hbm,
            tnx_hbm, tny_hbm, tnz_hbm, out_hbm,
            idx_v, d_v, gx_v, gy_v, gz_v, tx_v, ty_v, tz_v, acc_v, sem):
        wid = lax.axis_index("s") * NUM_CORES + lax.axis_index("c")
        base = wid * PER_TILE
        pltpu.sync_copy(i_hbm.at[pl.ds(base, PER_TILE)], idx_v)
        pltpu.sync_copy(d_hbm.at[pl.ds(base, PER_TILE)], d_v)
        pltpu.sync_copy(tnx_hbm.at[pl.ds(base, PER_TILE)], tx_v)
        pltpu.sync_copy(tny_hbm.at[pl.ds(base, PER_TILE)], ty_v)
        pltpu.sync_copy(tnz_hbm.at[pl.ds(base, PER_TILE)], tz_v)
        pltpu.async_copy(snx_hbm.at[idx_v], gx_v, sem).wait()
        pltpu.async_copy(sny_hbm.at[idx_v], gy_v, sem).wait()
        pltpu.async_copy(snz_hbm.at[idx_v], gz_v, sem).wait()
        acc = jnp.zeros((LANES,), jnp.float32)
        for j in range(PER_TILE // LANES):
            sl = pl.ds(j * LANES, LANES)
            dot = gx_v[sl] * tx_v[sl] + gy_v[sl] * ty_v[sl] + gz_v[sl] * tz_v[sl]
            acc = acc + jnp.where(dot > 0.5, d_v[sl], 0.0)
        acc_v[...] = acc
        pltpu.sync_copy(acc_v, out_hbm.at[wid])

    return sck(dists, idx, snx, sny, snz, tnx, tny, tnz)


def kernel(scan_vertices, template_vertices, scan_normals, template_normals):
    scan3 = jnp.pad(scan_vertices.T, ((0, 0), (0, NPAD - N_SCAN)),
                    constant_values=PAD_VAL).reshape(3, NROWS, 128)
    ssq = _ssq_call(scan3)
    dfold, ifold = _nn_call((template_vertices * -2.0).reshape(-1), scan3, ssq)
    d2, i2 = _reduce_call(dfold, ifold, template_vertices)
    dists = d2[:, 0]
    idx = i2[:, 0]
    snx, sny, snz = scan_normals[:, 0], scan_normals[:, 1], scan_normals[:, 2]
    tnx, tny, tnz = (template_normals[:, 0], template_normals[:, 1],
                     template_normals[:, 2])
    partials = _sc_gather_finish(dists, idx, snx, sny, snz, tnx, tny, tnz)
    return jnp.sum(partials)
